# baseline (speedup 1.0000x reference)
import functools

import jax
import jax.numpy as jnp
from jax import lax
from jax.experimental import pallas as pl
from jax.experimental.pallas import tpu as pltpu

N_TOK = 32768
D_MODEL = 768
N_EXPERTS = 64
TOP_K = 2
BLOCK_ROWS = 2048

_K0 = 0
_K1 = 42
_KS2 = 0x1BD11BDA ^ _K0 ^ _K1
_ROT_A = (13, 15, 26, 6)
_ROT_B = (17, 29, 16, 24)

_ERFINV_SMALL = (2.81022636e-08, 3.43273939e-07, -3.5233877e-06,
                 -4.39150654e-06, 0.00021858087, -0.00125372503,
                 -0.00417768164, 0.246640727, 1.50140941)
_ERFINV_BIG = (-0.000200214257, 0.000100950558, 0.00134934322,
               -0.00367342844, 0.00573950773, -0.0076224613,
               0.00943887047, 1.00167406, 2.83297682)


def _rotl(x, r):
    return lax.bitwise_or(lax.shift_left(x, jnp.int32(r)),
                          lax.shift_right_logical(x, jnp.int32(32 - r)))


def _threefry_bits(ctr):
    x0 = jnp.full_like(ctr, jnp.int32(_K0))
    x1 = ctr + jnp.int32(_K1)
    inject = ((_K1, _KS2), (_KS2, _K0), (_K0, _K1), (_K1, _KS2), (_KS2, _K0))
    for i in range(5):
        rots = _ROT_A if i % 2 == 0 else _ROT_B
        for r in rots:
            x0 = x0 + x1
            x1 = _rotl(x1, r)
            x1 = lax.bitwise_xor(x1, x0)
        x0 = x0 + jnp.int32(jnp.uint32(inject[i][0]).astype(jnp.int32))
        x1 = x1 + jnp.int32(jnp.uint32(inject[i][1]).astype(jnp.int32) + (i + 1))
    return lax.bitwise_xor(x0, x1)


def _bits_to_normal(bits):
    mant = lax.bitwise_or(lax.shift_right_logical(bits, jnp.int32(9)),
                          jnp.int32(0x3F800000))
    f01 = lax.bitcast_convert_type(mant, jnp.float32) - 1.0
    lo = jnp.float32(-0.99999994)
    u = f01 * (jnp.float32(1.0) - lo) + lo
    u = jnp.maximum(u, lo)
    w = -jnp.log1p(-u * u)
    ws = w - jnp.float32(2.5)
    wb = jnp.sqrt(w) - jnp.float32(3.0)
    ps = jnp.full_like(w, jnp.float32(_ERFINV_SMALL[0]))
    pb = jnp.full_like(w, jnp.float32(_ERFINV_BIG[0]))
    for c in _ERFINV_SMALL[1:]:
        ps = ps * ws + jnp.float32(c)
    for c in _ERFINV_BIG[1:]:
        pb = pb * wb + jnp.float32(c)
    p = jnp.where(w < jnp.float32(5.0), ps, pb)
    return jnp.float32(1.4142135381698608) * (p * u)


CHUNK_ROWS = 128


def _router_kernel(x_ref, wg_ref, bg_ref, wn_ref, bn_ref,
                   w_out_ref, idx_out_ref, mask_out_ref):
    row_base = pl.program_id(0) * BLOCK_ROWS
    wg = wg_ref[...]
    wn = wn_ref[...]
    bg = bg_ref[...]
    bn = bn_ref[...]

    shape = (CHUNK_ROWS, N_EXPERTS)
    lane = lax.broadcasted_iota(jnp.int32, shape, 1)
    lane_f = lane.astype(jnp.float32)
    rev = jnp.float32(N_EXPERTS - 1) - lane_f
    neg_inf = jnp.float32(-jnp.inf)

    half = CHUNK_ROWS // 2
    pshape = (half, 2 * N_EXPERTS)
    pr = lax.broadcasted_iota(jnp.int32, pshape, 0)
    pc = lax.broadcasted_iota(jnp.int32, pshape, 1)
    ctr0 = (pr * N_EXPERTS + pc
            + jnp.where(pc >= N_EXPERTS,
                        jnp.int32(half * N_EXPERTS - N_EXPERTS), jnp.int32(0)))

    for c in range(BLOCK_ROWS // CHUNK_ROWS):
        rows = pl.ds(c * CHUNK_ROWS, CHUNK_ROWS)
        x = x_ref[rows, :]
        logits = jnp.dot(x, wg, preferred_element_type=jnp.float32) + bg
        noise_in = jnp.dot(x, wn, preferred_element_type=jnp.float32) + bn

        row0 = row_base + c * CHUNK_ROWS
        ctr = ctr0 + row0 * N_EXPERTS
        eps_p = _bits_to_normal(_threefry_bits(ctr))
        eps = jnp.concatenate(
            [eps_p[:, :N_EXPERTS], eps_p[:, N_EXPERTS:]], axis=0)

        noisy = logits + eps * jax.nn.softplus(noise_in)

        v1 = jnp.max(noisy, axis=1, keepdims=True)
        m1 = jnp.max(jnp.where(noisy == v1, rev, neg_inf),
                     axis=1, keepdims=True)
        i1f = jnp.float32(N_EXPERTS - 1) - m1
        hot1 = lane_f == i1f
        masked = jnp.where(hot1, neg_inf, noisy)
        v2 = jnp.max(masked, axis=1, keepdims=True)
        m2 = jnp.max(jnp.where(masked == v2, rev, neg_inf),
                     axis=1, keepdims=True)
        i2f = jnp.float32(N_EXPERTS - 1) - m2
        hot2 = lane_f == i2f

        mask_out_ref[rows, :] = (hot1 | hot2).astype(jnp.float32)

        e2 = jnp.exp(v2 - v1)
        denom = 1.0 + e2
        w_out_ref[rows, :] = jnp.concatenate([1.0 / denom, e2 / denom], axis=1)
        idx_out_ref[rows, :] = jnp.concatenate(
            [i1f.astype(jnp.int32), i2f.astype(jnp.int32)], axis=1)


@functools.partial(jax.jit, static_argnames=())
def kernel(x, W_gate, b_gate, W_noise, b_noise):
    grid = (N_TOK // BLOCK_ROWS,)
    out_shapes = (
        jax.ShapeDtypeStruct((N_TOK, TOP_K), jnp.float32),
        jax.ShapeDtypeStruct((N_TOK, TOP_K), jnp.int32),
        jax.ShapeDtypeStruct((N_TOK, N_EXPERTS), jnp.float32),
    )
    weights, topk_idx, mask = pl.pallas_call(
        _router_kernel,
        grid=grid,
        in_specs=[
            pl.BlockSpec((BLOCK_ROWS, D_MODEL), lambda i: (i, 0)),
            pl.BlockSpec((D_MODEL, N_EXPERTS), lambda i: (0, 0)),
            pl.BlockSpec((N_EXPERTS,), lambda i: (0,)),
            pl.BlockSpec((D_MODEL, N_EXPERTS), lambda i: (0, 0)),
            pl.BlockSpec((N_EXPERTS,), lambda i: (0,)),
        ],
        out_specs=(
            pl.BlockSpec((BLOCK_ROWS, TOP_K), lambda i: (i, 0)),
            pl.BlockSpec((BLOCK_ROWS, TOP_K), lambda i: (i, 0)),
            pl.BlockSpec((BLOCK_ROWS, N_EXPERTS), lambda i: (i, 0)),
        ),
        out_shape=out_shapes,
        compiler_params=pltpu.CompilerParams(
            dimension_semantics=("parallel",)),
    )(x, W_gate, b_gate, W_noise, b_noise)
    return weights, topk_idx, mask


# speedup vs baseline: 1.0135x; 1.0135x over previous
import functools

import jax
import jax.numpy as jnp
from jax import lax
from jax.experimental import pallas as pl
from jax.experimental.pallas import tpu as pltpu

N_TOK = 32768
D_MODEL = 768
N_EXPERTS = 64
TOP_K = 2
BLOCK_ROWS = 4096

_K0 = 0
_K1 = 42
_KS2 = 0x1BD11BDA ^ _K0 ^ _K1
_ROT_A = (13, 15, 26, 6)
_ROT_B = (17, 29, 16, 24)

_ERFINV_SMALL = (2.81022636e-08, 3.43273939e-07, -3.5233877e-06,
                 -4.39150654e-06, 0.00021858087, -0.00125372503,
                 -0.00417768164, 0.246640727, 1.50140941)
_ERFINV_BIG = (-0.000200214257, 0.000100950558, 0.00134934322,
               -0.00367342844, 0.00573950773, -0.0076224613,
               0.00943887047, 1.00167406, 2.83297682)


def _rotl(x, r):
    return lax.bitwise_or(lax.shift_left(x, jnp.int32(r)),
                          lax.shift_right_logical(x, jnp.int32(32 - r)))


def _threefry_bits(ctr):
    x0 = jnp.full_like(ctr, jnp.int32(_K0))
    x1 = ctr + jnp.int32(_K1)
    inject = ((_K1, _KS2), (_KS2, _K0), (_K0, _K1), (_K1, _KS2), (_KS2, _K0))
    for i in range(5):
        rots = _ROT_A if i % 2 == 0 else _ROT_B
        for r in rots:
            x0 = x0 + x1
            x1 = _rotl(x1, r)
            x1 = lax.bitwise_xor(x1, x0)
        x0 = x0 + jnp.int32(jnp.uint32(inject[i][0]).astype(jnp.int32))
        x1 = x1 + jnp.int32(jnp.uint32(inject[i][1]).astype(jnp.int32) + (i + 1))
    return lax.bitwise_xor(x0, x1)


def _bits_to_normal(bits):
    mant = lax.bitwise_or(lax.shift_right_logical(bits, jnp.int32(9)),
                          jnp.int32(0x3F800000))
    f01 = lax.bitcast_convert_type(mant, jnp.float32) - 1.0
    lo = jnp.float32(-0.99999994)
    u = f01 * (jnp.float32(1.0) - lo) + lo
    u = jnp.maximum(u, lo)
    w = -jnp.log1p(-u * u)
    ws = w - jnp.float32(2.5)
    wb = jnp.sqrt(w) - jnp.float32(3.0)
    ps = jnp.full_like(w, jnp.float32(_ERFINV_SMALL[0]))
    pb = jnp.full_like(w, jnp.float32(_ERFINV_BIG[0]))
    for c in _ERFINV_SMALL[1:]:
        ps = ps * ws + jnp.float32(c)
    for c in _ERFINV_BIG[1:]:
        pb = pb * wb + jnp.float32(c)
    p = jnp.where(w < jnp.float32(5.0), ps, pb)
    return jnp.float32(1.4142135381698608) * (p * u)


CHUNK_ROWS = 256


def _router_kernel(x_ref, wg_ref, bg_ref, wn_ref, bn_ref,
                   w_out_ref, idx_out_ref, mask_out_ref):
    row_base = pl.program_id(0) * BLOCK_ROWS
    wg = wg_ref[...]
    wn = wn_ref[...]
    bg = bg_ref[...]
    bn = bn_ref[...]

    shape = (CHUNK_ROWS, N_EXPERTS)
    lane = lax.broadcasted_iota(jnp.int32, shape, 1)
    lane_f = lane.astype(jnp.float32)
    rev = jnp.float32(N_EXPERTS - 1) - lane_f
    neg_inf = jnp.float32(-jnp.inf)

    half = CHUNK_ROWS // 2
    pshape = (half, 2 * N_EXPERTS)
    pr = lax.broadcasted_iota(jnp.int32, pshape, 0)
    pc = lax.broadcasted_iota(jnp.int32, pshape, 1)
    ctr0 = (pr * N_EXPERTS + pc
            + jnp.where(pc >= N_EXPERTS,
                        jnp.int32(half * N_EXPERTS - N_EXPERTS), jnp.int32(0)))

    for c in range(BLOCK_ROWS // CHUNK_ROWS):
        rows = pl.ds(c * CHUNK_ROWS, CHUNK_ROWS)
        x = x_ref[rows, :]
        logits = jnp.dot(x, wg, preferred_element_type=jnp.float32) + bg
        noise_in = jnp.dot(x, wn, preferred_element_type=jnp.float32) + bn

        row0 = row_base + c * CHUNK_ROWS
        ctr = ctr0 + row0 * N_EXPERTS
        eps_p = _bits_to_normal(_threefry_bits(ctr))
        eps = jnp.concatenate(
            [eps_p[:, :N_EXPERTS], eps_p[:, N_EXPERTS:]], axis=0)

        noisy = logits + eps * jax.nn.softplus(noise_in)

        v1 = jnp.max(noisy, axis=1, keepdims=True)
        m1 = jnp.max(jnp.where(noisy == v1, rev, neg_inf),
                     axis=1, keepdims=True)
        i1f = jnp.float32(N_EXPERTS - 1) - m1
        hot1 = lane_f == i1f
        masked = jnp.where(hot1, neg_inf, noisy)
        v2 = jnp.max(masked, axis=1, keepdims=True)
        m2 = jnp.max(jnp.where(masked == v2, rev, neg_inf),
                     axis=1, keepdims=True)
        i2f = jnp.float32(N_EXPERTS - 1) - m2
        hot2 = lane_f == i2f

        mask_out_ref[rows, :] = (hot1 | hot2).astype(jnp.float32)

        e2 = jnp.exp(v2 - v1)
        denom = 1.0 + e2
        w_out_ref[rows, :] = jnp.concatenate([1.0 / denom, e2 / denom], axis=1)
        idx_out_ref[rows, :] = jnp.concatenate(
            [i1f.astype(jnp.int32), i2f.astype(jnp.int32)], axis=1)


@functools.partial(jax.jit, static_argnames=())
def kernel(x, W_gate, b_gate, W_noise, b_noise):
    grid = (N_TOK // BLOCK_ROWS,)
    out_shapes = (
        jax.ShapeDtypeStruct((N_TOK, TOP_K), jnp.float32),
        jax.ShapeDtypeStruct((N_TOK, TOP_K), jnp.int32),
        jax.ShapeDtypeStruct((N_TOK, N_EXPERTS), jnp.float32),
    )
    weights, topk_idx, mask = pl.pallas_call(
        _router_kernel,
        grid=grid,
        in_specs=[
            pl.BlockSpec((BLOCK_ROWS, D_MODEL), lambda i: (i, 0)),
            pl.BlockSpec((D_MODEL, N_EXPERTS), lambda i: (0, 0)),
            pl.BlockSpec((N_EXPERTS,), lambda i: (0,)),
            pl.BlockSpec((D_MODEL, N_EXPERTS), lambda i: (0, 0)),
            pl.BlockSpec((N_EXPERTS,), lambda i: (0,)),
        ],
        out_specs=(
            pl.BlockSpec((BLOCK_ROWS, TOP_K), lambda i: (i, 0)),
            pl.BlockSpec((BLOCK_ROWS, TOP_K), lambda i: (i, 0)),
            pl.BlockSpec((BLOCK_ROWS, N_EXPERTS), lambda i: (i, 0)),
        ),
        out_shape=out_shapes,
        compiler_params=pltpu.CompilerParams(
            dimension_semantics=("parallel",)),
    )(x, W_gate, b_gate, W_noise, b_noise)
    return weights, topk_idx, mask


# speedup vs baseline: 1.0403x; 1.0264x over previous
import functools

import jax
import jax.numpy as jnp
from jax import lax
from jax.experimental import pallas as pl
from jax.experimental.pallas import tpu as pltpu

N_TOK = 32768
D_MODEL = 768
N_EXPERTS = 64
TOP_K = 2
BLOCK_ROWS = 1024

_K0 = 0
_K1 = 42
_KS2 = 0x1BD11BDA ^ _K0 ^ _K1
_ROT_A = (13, 15, 26, 6)
_ROT_B = (17, 29, 16, 24)

_ERFINV_SMALL = (2.81022636e-08, 3.43273939e-07, -3.5233877e-06,
                 -4.39150654e-06, 0.00021858087, -0.00125372503,
                 -0.00417768164, 0.246640727, 1.50140941)
_ERFINV_BIG = (-0.000200214257, 0.000100950558, 0.00134934322,
               -0.00367342844, 0.00573950773, -0.0076224613,
               0.00943887047, 1.00167406, 2.83297682)


def _rotl(x, r):
    return lax.bitwise_or(lax.shift_left(x, jnp.int32(r)),
                          lax.shift_right_logical(x, jnp.int32(32 - r)))


def _threefry_bits(ctr):
    x0 = jnp.full_like(ctr, jnp.int32(_K0))
    x1 = ctr + jnp.int32(_K1)
    inject = ((_K1, _KS2), (_KS2, _K0), (_K0, _K1), (_K1, _KS2), (_KS2, _K0))
    for i in range(5):
        rots = _ROT_A if i % 2 == 0 else _ROT_B
        for r in rots:
            x0 = x0 + x1
            x1 = _rotl(x1, r)
            x1 = lax.bitwise_xor(x1, x0)
        x0 = x0 + jnp.int32(jnp.uint32(inject[i][0]).astype(jnp.int32))
        x1 = x1 + jnp.int32(jnp.uint32(inject[i][1]).astype(jnp.int32) + (i + 1))
    return lax.bitwise_xor(x0, x1)


def _bits_to_normal(bits):
    mant = lax.bitwise_or(lax.shift_right_logical(bits, jnp.int32(9)),
                          jnp.int32(0x3F800000))
    f01 = lax.bitcast_convert_type(mant, jnp.float32) - 1.0
    lo = jnp.float32(-0.99999994)
    u = f01 * (jnp.float32(1.0) - lo) + lo
    u = jnp.maximum(u, lo)
    w = -jnp.log1p(-u * u)
    ws = w - jnp.float32(2.5)
    wb = jnp.sqrt(w) - jnp.float32(3.0)
    ps = jnp.full_like(w, jnp.float32(_ERFINV_SMALL[0]))
    pb = jnp.full_like(w, jnp.float32(_ERFINV_BIG[0]))
    for c in _ERFINV_SMALL[1:]:
        ps = ps * ws + jnp.float32(c)
    for c in _ERFINV_BIG[1:]:
        pb = pb * wb + jnp.float32(c)
    p = jnp.where(w < jnp.float32(5.0), ps, pb)
    return jnp.float32(1.4142135381698608) * (p * u)


CHUNK_ROWS = 256


def _router_kernel(x_ref, wg_ref, bg_ref, wn_ref, bn_ref,
                   w_out_ref, idx_out_ref, mask_out_ref):
    row_base = pl.program_id(0) * BLOCK_ROWS
    wg = wg_ref[...]
    wn = wn_ref[...]
    bg = bg_ref[...]
    bn = bn_ref[...]

    shape = (CHUNK_ROWS, N_EXPERTS)
    lane = lax.broadcasted_iota(jnp.int32, shape, 1)
    lane_f = lane.astype(jnp.float32)
    rev = jnp.float32(N_EXPERTS - 1) - lane_f
    neg_inf = jnp.float32(-jnp.inf)

    half = CHUNK_ROWS // 2
    pshape = (half, 2 * N_EXPERTS)
    pr = lax.broadcasted_iota(jnp.int32, pshape, 0)
    pc = lax.broadcasted_iota(jnp.int32, pshape, 1)
    ctr0 = (pr * N_EXPERTS + pc
            + jnp.where(pc >= N_EXPERTS,
                        jnp.int32(half * N_EXPERTS - N_EXPERTS), jnp.int32(0)))

    for c in range(BLOCK_ROWS // CHUNK_ROWS):
        rows = pl.ds(c * CHUNK_ROWS, CHUNK_ROWS)
        x = x_ref[rows, :]
        logits = jnp.dot(x, wg, preferred_element_type=jnp.float32) + bg
        noise_in = jnp.dot(x, wn, preferred_element_type=jnp.float32) + bn

        row0 = row_base + c * CHUNK_ROWS
        ctr = ctr0 + row0 * N_EXPERTS
        eps_p = _bits_to_normal(_threefry_bits(ctr))
        eps = jnp.concatenate(
            [eps_p[:, :N_EXPERTS], eps_p[:, N_EXPERTS:]], axis=0)

        noisy = logits + eps * jax.nn.softplus(noise_in)

        v1 = jnp.max(noisy, axis=1, keepdims=True)
        m1 = jnp.max(jnp.where(noisy == v1, rev, neg_inf),
                     axis=1, keepdims=True)
        i1f = jnp.float32(N_EXPERTS - 1) - m1
        hot1 = lane_f == i1f
        masked = jnp.where(hot1, neg_inf, noisy)
        v2 = jnp.max(masked, axis=1, keepdims=True)
        m2 = jnp.max(jnp.where(masked == v2, rev, neg_inf),
                     axis=1, keepdims=True)
        i2f = jnp.float32(N_EXPERTS - 1) - m2
        hot2 = lane_f == i2f

        mask_out_ref[rows, :] = (hot1 | hot2).astype(jnp.float32)

        e2 = jnp.exp(v2 - v1)
        denom = 1.0 + e2
        w_out_ref[rows, :] = jnp.concatenate([1.0 / denom, e2 / denom], axis=1)
        idx_out_ref[rows, :] = jnp.concatenate(
            [i1f.astype(jnp.int32), i2f.astype(jnp.int32)], axis=1)


@functools.partial(jax.jit, static_argnames=())
def kernel(x, W_gate, b_gate, W_noise, b_noise):
    grid = (N_TOK // BLOCK_ROWS,)
    out_shapes = (
        jax.ShapeDtypeStruct((N_TOK, TOP_K), jnp.float32),
        jax.ShapeDtypeStruct((N_TOK, TOP_K), jnp.int32),
        jax.ShapeDtypeStruct((N_TOK, N_EXPERTS), jnp.float32),
    )
    weights, topk_idx, mask = pl.pallas_call(
        _router_kernel,
        grid=grid,
        in_specs=[
            pl.BlockSpec((BLOCK_ROWS, D_MODEL), lambda i: (i, 0)),
            pl.BlockSpec((D_MODEL, N_EXPERTS), lambda i: (0, 0)),
            pl.BlockSpec((N_EXPERTS,), lambda i: (0,)),
            pl.BlockSpec((D_MODEL, N_EXPERTS), lambda i: (0, 0)),
            pl.BlockSpec((N_EXPERTS,), lambda i: (0,)),
        ],
        out_specs=(
            pl.BlockSpec((BLOCK_ROWS, TOP_K), lambda i: (i, 0)),
            pl.BlockSpec((BLOCK_ROWS, TOP_K), lambda i: (i, 0)),
            pl.BlockSpec((BLOCK_ROWS, N_EXPERTS), lambda i: (i, 0)),
        ),
        out_shape=out_shapes,
        compiler_params=pltpu.CompilerParams(
            dimension_semantics=("parallel",)),
    )(x, W_gate, b_gate, W_noise, b_noise)
    return weights, topk_idx, mask


# speedup vs baseline: 1.0879x; 1.0457x over previous
import functools

import jax
import jax.numpy as jnp
from jax import lax
from jax.experimental import pallas as pl
from jax.experimental.pallas import tpu as pltpu

N_TOK = 32768
D_MODEL = 768
N_EXPERTS = 64
TOP_K = 2
BLOCK_ROWS = 2048

_K0 = 0
_K1 = 42
_KS2 = 0x1BD11BDA ^ _K0 ^ _K1
_ROT_A = (13, 15, 26, 6)
_ROT_B = (17, 29, 16, 24)

_ERFINV_SMALL = (2.81022636e-08, 3.43273939e-07, -3.5233877e-06,
                 -4.39150654e-06, 0.00021858087, -0.00125372503,
                 -0.00417768164, 0.246640727, 1.50140941)
_ERFINV_BIG = (-0.000200214257, 0.000100950558, 0.00134934322,
               -0.00367342844, 0.00573950773, -0.0076224613,
               0.00943887047, 1.00167406, 2.83297682)


def _rotl(x, r):
    return lax.bitwise_or(lax.shift_left(x, jnp.int32(r)),
                          lax.shift_right_logical(x, jnp.int32(32 - r)))


def _threefry_bits(x1):
    x0 = jnp.full_like(x1, jnp.int32(_K0))
    inject = ((_K1, _KS2), (_KS2, _K0), (_K0, _K1), (_K1, _KS2), (_KS2, _K0))
    for i in range(5):
        rots = _ROT_A if i % 2 == 0 else _ROT_B
        for r in rots:
            x0 = x0 + x1
            x1 = _rotl(x1, r)
            x1 = lax.bitwise_xor(x1, x0)
        x0 = x0 + jnp.int32(jnp.uint32(inject[i][0]).astype(jnp.int32))
        x1 = x1 + jnp.int32(jnp.uint32(inject[i][1]).astype(jnp.int32) + (i + 1))
    return lax.bitwise_xor(x0, x1)


def _bits_to_normal(bits):
    mant = lax.bitwise_or(lax.shift_right_logical(bits, jnp.int32(9)),
                          jnp.int32(0x3F800000))
    f01 = lax.bitcast_convert_type(mant, jnp.float32) - 1.0
    lo = jnp.float32(-0.99999994)
    u = f01 * (jnp.float32(1.0) - lo) + lo
    w = -jnp.log1p(-u * u)
    ws = w - jnp.float32(2.5)
    wb = jnp.sqrt(w) - jnp.float32(3.0)
    ps = jnp.full_like(w, jnp.float32(_ERFINV_SMALL[0]))
    pb = jnp.full_like(w, jnp.float32(_ERFINV_BIG[0]))
    for c in _ERFINV_SMALL[1:]:
        ps = ps * ws + jnp.float32(c)
    for c in _ERFINV_BIG[1:]:
        pb = pb * wb + jnp.float32(c)
    p = jnp.where(w < jnp.float32(5.0), ps, pb)
    return jnp.float32(1.4142135381698608) * (p * u)


CHUNK_ROWS = 256


def _router_kernel(x_ref, wg_ref, bg_ref, wn_ref, bn_ref,
                   w_out_ref, idx_out_ref, mask_out_ref):
    row_base = pl.program_id(0) * BLOCK_ROWS
    wg = wg_ref[...]
    wn = wn_ref[...]
    bg = bg_ref[...]
    bn = bn_ref[...]

    shape = (CHUNK_ROWS, N_EXPERTS)
    lane = lax.broadcasted_iota(jnp.int32, shape, 1)
    lane_f = lane.astype(jnp.float32)
    rev = jnp.float32(N_EXPERTS - 1) - lane_f
    neg_inf = jnp.float32(-jnp.inf)

    half = CHUNK_ROWS // 2
    pshape = (half, 2 * N_EXPERTS)
    pr = lax.broadcasted_iota(jnp.int32, pshape, 0)
    pc = lax.broadcasted_iota(jnp.int32, pshape, 1)
    ctr0 = (pr * N_EXPERTS + pc + jnp.int32(_K1)
            + jnp.where(pc >= N_EXPERTS,
                        jnp.int32(half * N_EXPERTS - N_EXPERTS), jnp.int32(0)))

    for c in range(BLOCK_ROWS // CHUNK_ROWS):
        rows = pl.ds(c * CHUNK_ROWS, CHUNK_ROWS)
        row0 = row_base + c * CHUNK_ROWS
        ctr = ctr0 + row0 * N_EXPERTS
        eps_p = _bits_to_normal(_threefry_bits(ctr))
        eps = jnp.concatenate(
            [eps_p[:, :N_EXPERTS], eps_p[:, N_EXPERTS:]], axis=0)

        x = x_ref[rows, :]
        logits = jnp.dot(x, wg, preferred_element_type=jnp.float32) + bg
        noise_in = jnp.dot(x, wn, preferred_element_type=jnp.float32) + bn

        noisy = logits + eps * jax.nn.softplus(noise_in)

        v1 = jnp.max(noisy, axis=1, keepdims=True)
        m1 = jnp.max(jnp.where(noisy == v1, rev, neg_inf),
                     axis=1, keepdims=True)
        i1f = jnp.float32(N_EXPERTS - 1) - m1
        hot1 = lane_f == i1f
        masked = jnp.where(hot1, neg_inf, noisy)
        v2 = jnp.max(masked, axis=1, keepdims=True)
        m2 = jnp.max(jnp.where(masked == v2, rev, neg_inf),
                     axis=1, keepdims=True)
        i2f = jnp.float32(N_EXPERTS - 1) - m2
        hot2 = lane_f == i2f

        mask_out_ref[rows, :] = (hot1 | hot2).astype(jnp.float32)

        e2 = jnp.exp(v2 - v1)
        denom = 1.0 + e2
        w_out_ref[rows, :] = jnp.concatenate([1.0 / denom, e2 / denom], axis=1)
        idx_out_ref[rows, :] = jnp.concatenate(
            [i1f.astype(jnp.int32), i2f.astype(jnp.int32)], axis=1)


@functools.partial(jax.jit, static_argnames=())
def kernel(x, W_gate, b_gate, W_noise, b_noise):
    grid = (N_TOK // BLOCK_ROWS,)
    out_shapes = (
        jax.ShapeDtypeStruct((N_TOK, TOP_K), jnp.float32),
        jax.ShapeDtypeStruct((N_TOK, TOP_K), jnp.int32),
        jax.ShapeDtypeStruct((N_TOK, N_EXPERTS), jnp.float32),
    )
    weights, topk_idx, mask = pl.pallas_call(
        _router_kernel,
        grid=grid,
        in_specs=[
            pl.BlockSpec((BLOCK_ROWS, D_MODEL), lambda i: (i, 0)),
            pl.BlockSpec((D_MODEL, N_EXPERTS), lambda i: (0, 0)),
            pl.BlockSpec((N_EXPERTS,), lambda i: (0,)),
            pl.BlockSpec((D_MODEL, N_EXPERTS), lambda i: (0, 0)),
            pl.BlockSpec((N_EXPERTS,), lambda i: (0,)),
        ],
        out_specs=(
            pl.BlockSpec((BLOCK_ROWS, TOP_K), lambda i: (i, 0)),
            pl.BlockSpec((BLOCK_ROWS, TOP_K), lambda i: (i, 0)),
            pl.BlockSpec((BLOCK_ROWS, N_EXPERTS), lambda i: (i, 0)),
        ),
        out_shape=out_shapes,
        compiler_params=pltpu.CompilerParams(
            dimension_semantics=("parallel",)),
    )(x, W_gate, b_gate, W_noise, b_noise)
    return weights, topk_idx, mask


# speedup vs baseline: 1.1170x; 1.0268x over previous
import functools

import jax
import jax.numpy as jnp
from jax import lax
from jax.experimental import pallas as pl
from jax.experimental.pallas import tpu as pltpu

N_TOK = 32768
D_MODEL = 768
N_EXPERTS = 64
TOP_K = 2
BLOCK_ROWS = 2048

_K0 = 0
_K1 = 42
_KS2 = 0x1BD11BDA ^ _K0 ^ _K1
_ROT_A = (13, 15, 26, 6)
_ROT_B = (17, 29, 16, 24)

_ERFINV_SMALL = (2.81022636e-08, 3.43273939e-07, -3.5233877e-06,
                 -4.39150654e-06, 0.00021858087, -0.00125372503,
                 -0.00417768164, 0.246640727, 1.50140941)
_ERFINV_BIG = (-0.000200214257, 0.000100950558, 0.00134934322,
               -0.00367342844, 0.00573950773, -0.0076224613,
               0.00943887047, 1.00167406, 2.83297682)


def _rotl(x, r):
    return lax.bitwise_or(lax.shift_left(x, jnp.int32(r)),
                          lax.shift_right_logical(x, jnp.int32(32 - r)))


def _threefry_bits(x1):
    x0 = jnp.full_like(x1, jnp.int32(_K0))
    inject = ((_K1, _KS2), (_KS2, _K0), (_K0, _K1), (_K1, _KS2), (_KS2, _K0))
    for i in range(5):
        rots = _ROT_A if i % 2 == 0 else _ROT_B
        for r in rots:
            x0 = x0 + x1
            x1 = _rotl(x1, r)
            x1 = lax.bitwise_xor(x1, x0)
        x0 = x0 + jnp.int32(jnp.uint32(inject[i][0]).astype(jnp.int32))
        x1 = x1 + jnp.int32(jnp.uint32(inject[i][1]).astype(jnp.int32) + (i + 1))
    return lax.bitwise_xor(x0, x1)


def _bits_to_normal(bits):
    mant = lax.bitwise_or(lax.shift_right_logical(bits, jnp.int32(9)),
                          jnp.int32(0x3F800000))
    f01 = lax.bitcast_convert_type(mant, jnp.float32) - 1.0
    lo = jnp.float32(-0.99999994)
    u = f01 * (jnp.float32(1.0) - lo) + lo
    w = -jnp.log(1.0 - u * u)
    ws = w - jnp.float32(2.5)
    wb = lax.rsqrt(w) * w - jnp.float32(3.0)
    ps = jnp.full_like(w, jnp.float32(_ERFINV_SMALL[0]))
    pb = jnp.full_like(w, jnp.float32(_ERFINV_BIG[0]))
    for c in _ERFINV_SMALL[1:]:
        ps = ps * ws + jnp.float32(c)
    for c in _ERFINV_BIG[1:]:
        pb = pb * wb + jnp.float32(c)
    p = jnp.where(w < jnp.float32(5.0), ps, pb)
    return jnp.float32(1.4142135381698608) * (p * u)


CHUNK_ROWS = 256


def _router_kernel(x_ref, wg_ref, bg_ref, wn_ref, bn_ref,
                   w_out_ref, idx_out_ref, mask_out_ref):
    row_base = pl.program_id(0) * BLOCK_ROWS
    wg = wg_ref[...]
    wn = wn_ref[...]
    bg = bg_ref[...]
    bn = bn_ref[...]

    shape = (CHUNK_ROWS, N_EXPERTS)
    lane = lax.broadcasted_iota(jnp.int32, shape, 1)
    lane_f = lane.astype(jnp.float32)
    rev = jnp.float32(N_EXPERTS - 1) - lane_f
    neg_inf = jnp.float32(-jnp.inf)

    half = CHUNK_ROWS // 2
    pshape = (half, 2 * N_EXPERTS)
    pr = lax.broadcasted_iota(jnp.int32, pshape, 0)
    pc = lax.broadcasted_iota(jnp.int32, pshape, 1)
    ctr0 = (pr * N_EXPERTS + pc + jnp.int32(_K1)
            + jnp.where(pc >= N_EXPERTS,
                        jnp.int32(half * N_EXPERTS - N_EXPERTS), jnp.int32(0)))

    for c in range(BLOCK_ROWS // CHUNK_ROWS):
        rows = pl.ds(c * CHUNK_ROWS, CHUNK_ROWS)
        row0 = row_base + c * CHUNK_ROWS
        ctr = ctr0 + row0 * N_EXPERTS
        eps_p = _bits_to_normal(_threefry_bits(ctr))
        eps = jnp.concatenate(
            [eps_p[:, :N_EXPERTS], eps_p[:, N_EXPERTS:]], axis=0)

        x = x_ref[rows, :]
        logits = jnp.dot(x, wg, preferred_element_type=jnp.float32) + bg
        noise_in = jnp.dot(x, wn, preferred_element_type=jnp.float32) + bn

        noisy = logits + eps * jax.nn.softplus(noise_in)

        v1 = jnp.max(noisy, axis=1, keepdims=True)
        m1 = jnp.max(jnp.where(noisy == v1, rev, neg_inf),
                     axis=1, keepdims=True)
        i1f = jnp.float32(N_EXPERTS - 1) - m1
        hot1 = lane_f == i1f
        masked = jnp.where(hot1, neg_inf, noisy)
        v2 = jnp.max(masked, axis=1, keepdims=True)
        m2 = jnp.max(jnp.where(masked == v2, rev, neg_inf),
                     axis=1, keepdims=True)
        i2f = jnp.float32(N_EXPERTS - 1) - m2
        hot2 = lane_f == i2f

        mask_out_ref[rows, :] = (hot1 | hot2).astype(jnp.float32)

        e2 = jnp.exp(v2 - v1)
        denom = 1.0 + e2
        w_out_ref[rows, :] = jnp.concatenate([1.0 / denom, e2 / denom], axis=1)
        idx_out_ref[rows, :] = jnp.concatenate(
            [i1f.astype(jnp.int32), i2f.astype(jnp.int32)], axis=1)


@functools.partial(jax.jit, static_argnames=())
def kernel(x, W_gate, b_gate, W_noise, b_noise):
    grid = (N_TOK // BLOCK_ROWS,)
    out_shapes = (
        jax.ShapeDtypeStruct((N_TOK, TOP_K), jnp.float32),
        jax.ShapeDtypeStruct((N_TOK, TOP_K), jnp.int32),
        jax.ShapeDtypeStruct((N_TOK, N_EXPERTS), jnp.float32),
    )
    weights, topk_idx, mask = pl.pallas_call(
        _router_kernel,
        grid=grid,
        in_specs=[
            pl.BlockSpec((BLOCK_ROWS, D_MODEL), lambda i: (i, 0)),
            pl.BlockSpec((D_MODEL, N_EXPERTS), lambda i: (0, 0)),
            pl.BlockSpec((N_EXPERTS,), lambda i: (0,)),
            pl.BlockSpec((D_MODEL, N_EXPERTS), lambda i: (0, 0)),
            pl.BlockSpec((N_EXPERTS,), lambda i: (0,)),
        ],
        out_specs=(
            pl.BlockSpec((BLOCK_ROWS, TOP_K), lambda i: (i, 0)),
            pl.BlockSpec((BLOCK_ROWS, TOP_K), lambda i: (i, 0)),
            pl.BlockSpec((BLOCK_ROWS, N_EXPERTS), lambda i: (i, 0)),
        ),
        out_shape=out_shapes,
        compiler_params=pltpu.CompilerParams(
            dimension_semantics=("parallel",)),
    )(x, W_gate, b_gate, W_noise, b_noise)
    return weights, topk_idx, mask
